# Initial kernel scaffold; baseline (speedup 1.0000x reference)
import jax
import jax.numpy as jnp
from jax.experimental import pallas as pl


def kernel(x, edge_index, Wl1, bl1, Wr1, Wl2, bl2, Wr2):
    raise NotImplementedError("write your pallas kernel here")


# speedup vs baseline: 4.2728x; 4.2728x over previous
import functools

import jax
import jax.numpy as jnp
from jax import lax
from jax.experimental import pallas as pl
from jax.experimental.pallas import tpu as pltpu
from jax.experimental.pallas import tpu_sc as plsc

N = 10000
E = 320000
D = 128
NC = 2
NS = 16
NW = NC * NS
K = 128
ET = -(-E // (NW * K)) * K
CT = ET // K
EPAD = ET * NW
NPAD = 10240
RPT = NPAD // NS

@functools.cache
def _mesh():
    return plsc.VectorSubcoreMesh(core_axis_name="c", subcore_axis_name="s",
                                  num_cores=NC, num_subcores=NS)


def _zero_vmem_2d(buf):
    def zrow(r, carry):
        for cc in range(D // 16):
            buf[r, pl.ds(cc * 16, 16)] = jnp.zeros((16,), jnp.float32)
        return carry
    lax.fori_loop(0, K, zrow, 0)


@functools.cache
def _sc_agg_cnt_kernel():
    return pl.kernel(
        _sc_agg_cnt_body,
        out_type=[
            jax.ShapeDtypeStruct((NPAD, D), jnp.float32),
            jax.ShapeDtypeStruct((NPAD, D), jnp.float32),
            jax.ShapeDtypeStruct((NPAD,), jnp.float32),
            jax.ShapeDtypeStruct((NPAD,), jnp.float32),
        ],
        mesh=_mesh(),
        scratch_types=[
            pltpu.VMEM_SHARED((NPAD, D), jnp.float32),
            pltpu.VMEM_SHARED((NPAD,), jnp.float32),
            pltpu.VMEM((K,), jnp.int32),
            pltpu.VMEM((K,), jnp.int32),
            pltpu.VMEM((K, D), jnp.float32),
            pltpu.VMEM((K, D), jnp.float32),
            pltpu.VMEM((RPT,), jnp.float32),
            pltpu.VMEM((K,), jnp.float32),
            pltpu.SemaphoreType.DMA,
        ],
    )


def _sc_agg_cnt_body(x_hbm, src_hbm, dst_hbm, acc0, acc1, cnt0, cnt1,
                acc_sh, cnt_sh, idx_s, idx_d, rows, zbuf, zcnt, ones_v, sem):
    c = lax.axis_index("c")
    s = lax.axis_index("s")

    _zero_vmem_2d(zbuf)
    for j in range(RPT // K):
        pltpu.sync_copy(zbuf, acc_sh.at[pl.ds(s * RPT + j * K, K)])

    def zfill(r, carry):
        zcnt[pl.ds(r * 16, 16)] = jnp.zeros((16,), jnp.float32)
        return carry
    lax.fori_loop(0, RPT // 16, zfill, 0)
    for cc in range(K // 16):
        ones_v[pl.ds(cc * 16, 16)] = jnp.ones((16,), jnp.float32)
    pltpu.sync_copy(zcnt, cnt_sh.at[pl.ds(s * RPT, RPT)])
    plsc.subcore_barrier()

    base = (c * NS + s) * ET

    def chunk(i, carry):
        off = base + i * K
        pltpu.sync_copy(src_hbm.at[pl.ds(off, K)], idx_s)
        pltpu.sync_copy(dst_hbm.at[pl.ds(off, K)], idx_d)
        pltpu.async_copy(x_hbm.at[idx_s], rows, sem).wait()
        pltpu.sync_copy(rows, acc_sh.at[idx_d], add=True)
        pltpu.sync_copy(ones_v, cnt_sh.at[idx_d], add=True)
        return carry
    lax.fori_loop(0, CT, chunk, 0)

    plsc.subcore_barrier()

    rb = s * RPT

    @pl.when(c == 0)
    def _():
        pltpu.sync_copy(acc_sh.at[pl.ds(rb, RPT)], acc0.at[pl.ds(rb, RPT)])
        pltpu.sync_copy(cnt_sh.at[pl.ds(rb, RPT)], cnt0.at[pl.ds(rb, RPT)])

    @pl.when(c == 1)
    def _():
        pltpu.sync_copy(acc_sh.at[pl.ds(rb, RPT)], acc1.at[pl.ds(rb, RPT)])
        pltpu.sync_copy(cnt_sh.at[pl.ds(rb, RPT)], cnt1.at[pl.ds(rb, RPT)])


@functools.cache
def _sc_agg_kernel():
    return pl.kernel(
        _sc_agg_body,
        out_type=[
            jax.ShapeDtypeStruct((NPAD, D), jnp.float32),
            jax.ShapeDtypeStruct((NPAD, D), jnp.float32),
        ],
        mesh=_mesh(),
        scratch_types=[
            pltpu.VMEM_SHARED((NPAD, D), jnp.float32),
            pltpu.VMEM((K,), jnp.int32),
            pltpu.VMEM((K,), jnp.int32),
            pltpu.VMEM((K, D), jnp.float32),
            pltpu.VMEM((K, D), jnp.float32),
            pltpu.SemaphoreType.DMA,
        ],
    )


def _sc_agg_body(x_hbm, src_hbm, dst_hbm, acc0, acc1,
            acc_sh, idx_s, idx_d, rows, zbuf, sem):
    c = lax.axis_index("c")
    s = lax.axis_index("s")

    _zero_vmem_2d(zbuf)
    for j in range(RPT // K):
        pltpu.sync_copy(zbuf, acc_sh.at[pl.ds(s * RPT + j * K, K)])
    plsc.subcore_barrier()

    base = (c * NS + s) * ET

    def chunk(i, carry):
        off = base + i * K
        pltpu.sync_copy(src_hbm.at[pl.ds(off, K)], idx_s)
        pltpu.sync_copy(dst_hbm.at[pl.ds(off, K)], idx_d)
        pltpu.async_copy(x_hbm.at[idx_s], rows, sem).wait()
        pltpu.sync_copy(rows, acc_sh.at[idx_d], add=True)
        return carry
    lax.fori_loop(0, CT, chunk, 0)

    plsc.subcore_barrier()

    rb = s * RPT

    @pl.when(c == 0)
    def _():
        pltpu.sync_copy(acc_sh.at[pl.ds(rb, RPT)], acc0.at[pl.ds(rb, RPT)])

    @pl.when(c == 1)
    def _():
        pltpu.sync_copy(acc_sh.at[pl.ds(rb, RPT)], acc1.at[pl.ds(rb, RPT)])


def _tc_body(relu, a0_ref, a1_ref, c0_ref, c1_ref, x_ref, wl_ref, bl_ref,
             wr_ref, o_ref):
    agg = a0_ref[...] + a1_ref[...]
    inv = 1.0 / jnp.maximum(c0_ref[...] + c1_ref[...], 1.0)
    z = (jnp.dot(agg * inv, wl_ref[...], preferred_element_type=jnp.float32)
         + bl_ref[...]
         + jnp.dot(x_ref[...], wr_ref[...], preferred_element_type=jnp.float32))
    o_ref[...] = jnp.maximum(z, 0.0) if relu else z


def _tc_layer(relu, out_rows, block_rows, acc0, acc1, cnt0, cnt1, x, wlt,
              bl, wrt):
    grid = out_rows // block_rows
    row_spec = pl.BlockSpec((block_rows, D), lambda i: (i, 0))
    cnt_spec = pl.BlockSpec((block_rows, 1), lambda i: (i, 0))
    full = pl.BlockSpec((D, D), lambda i: (0, 0))
    bspec = pl.BlockSpec((1, D), lambda i: (0, 0))
    return pl.pallas_call(
        functools.partial(_tc_body, relu),
        grid=(grid,),
        in_specs=[row_spec, row_spec, cnt_spec, cnt_spec, row_spec, full,
                  bspec, full],
        out_specs=row_spec,
        out_shape=jax.ShapeDtypeStruct((out_rows, D), jnp.float32),
    )(acc0, acc1, cnt0, cnt1, x, wlt, bl, wrt)


def kernel(x, edge_index, Wl1, bl1, Wr1, Wl2, bl2, Wr2):
    src = edge_index[0].astype(jnp.int32)
    dst = edge_index[1].astype(jnp.int32)
    pad = EPAD - E
    src_pad = jnp.concatenate([src, jnp.full((pad,), N, jnp.int32)])
    dst_pad = jnp.concatenate([dst, jnp.full((pad,), N, jnp.int32)])
    x_pad = jnp.concatenate([x, jnp.zeros((NPAD - N, D), x.dtype)])

    acc0, acc1, cnt0, cnt1 = _sc_agg_cnt_kernel()(x_pad, src_pad, dst_pad)
    cnt0 = cnt0.reshape(NPAD, 1)
    cnt1 = cnt1.reshape(NPAD, 1)

    h_pad = _tc_layer(True, NPAD, 512, acc0, acc1, cnt0, cnt1, x_pad,
                      Wl1.T, bl1.reshape(1, D), Wr1.T)

    b0, b1 = _sc_agg_kernel()(h_pad, src_pad, dst_pad)

    out = _tc_layer(False, N, 400, b0, b1, cnt0, cnt1, h_pad,
                    Wl2.T, bl2.reshape(1, D), Wr2.T)
    return out
